# baseline (speedup 1.0000x reference)
import functools

import jax
import jax.numpy as jnp
import numpy as np
from jax import lax
from jax.experimental import pallas as pl
from jax.experimental.pallas import tpu as pltpu
from jax.experimental.pallas import tpu_sc as plsc

N_FIELDS = 26
VOCAB_PER_FIELD = 100000
BATCH = 4096
EMBED_DIM = 64
TOTAL_ROWS = N_FIELDS * VOCAB_PER_FIELD
N_IDX = BATCH * N_FIELDS

GAMMA = 2000.0
PRETRAIN_EPOCH = 5
_TEMP = float(GAMMA ** (1.0 / (PRETRAIN_EPOCH - 1)))
_SCALING = float(1.0 + np.exp(-0.5))

_NC = 2
_NS = 16
_NW = _NC * _NS
_LANES = 16

_F_SPLIT = 24


def _make_sc_gather(first_field, n_fields):
    n_idx = n_fields * BATCH
    chunk = n_idx // _NW
    vecs = chunk // _LANES

    def body(raw_hbm, table_hbm, out_hbm, idx_v, rows_v, sem):
        wid = lax.axis_index("s") * _NC + lax.axis_index("c")
        base = wid * chunk
        pltpu.sync_copy(
            raw_hbm.at[pl.ds(first_field * BATCH + base, chunk)], idx_v)

        lane = lax.iota(jnp.int32, _LANES)

        @pl.loop(0, vecs, unroll=8)
        def _(i):
            s = pl.ds(i * _LANES, _LANES)
            pos = base + i * _LANES + lane
            field = lax.shift_right_logical(pos, 12)
            idx_v[s] = idx_v[s] + field * VOCAB_PER_FIELD

        pltpu.async_copy(table_hbm.at[idx_v], rows_v, sem).wait()
        pltpu.sync_copy(rows_v, out_hbm.at[pl.ds(base, chunk)])

    return functools.partial(
        pl.kernel,
        out_type=jax.ShapeDtypeStruct((n_idx,), jnp.float32),
        mesh=plsc.VectorSubcoreMesh(
            core_axis_name="c", subcore_axis_name="s", num_cores=_NC,
            num_subcores=_NS,
        ),
        scratch_types=[
            pltpu.VMEM((chunk,), jnp.int32),
            pltpu.VMEM((chunk,), jnp.float32),
            pltpu.SemaphoreType.DMA,
        ],
    )(body)


_sc_gather_a = _make_sc_gather(0, _F_SPLIT)
_sc_gather_b = _make_sc_gather(_F_SPLIT, N_FIELDS - _F_SPLIT)

_F_BLK = 2
_BLK_SPLIT = _F_SPLIT // _F_BLK


def _tc_mul_body(x_ref, mwa_ref, mwb_ref, o_ref):
    f = pl.program_id(0)
    mw = jnp.where(f < _BLK_SPLIT, mwa_ref[...], mwb_ref[...])
    gate = _SCALING * jax.nn.sigmoid(_TEMP * mw)
    o_ref[...] = x_ref[...] * gate.reshape(_F_BLK, 1, BATCH)


_tc_mul = pl.pallas_call(
    _tc_mul_body,
    grid=(N_FIELDS // _F_BLK,),
    in_specs=[
        pl.BlockSpec((_F_BLK, EMBED_DIM, BATCH), lambda f: (f, 0, 0)),
        pl.BlockSpec((_F_BLK * BATCH,),
                     lambda f: (jnp.minimum(f, _BLK_SPLIT - 1),)),
        pl.BlockSpec((_F_BLK * BATCH,),
                     lambda f: (jnp.maximum(f - _BLK_SPLIT, 0),)),
    ],
    out_specs=pl.BlockSpec((_F_BLK, EMBED_DIM, BATCH), lambda f: (f, 0, 0)),
    out_shape=jax.ShapeDtypeStruct((N_FIELDS, EMBED_DIM, BATCH), jnp.float32),
)


def kernel(x, current_epoch, current_step, raw_data, mask_weight):
    xt = jnp.transpose(x, (1, 2, 0))
    raw_flat = jnp.transpose(raw_data, (1, 0)).astype(jnp.int32).reshape(-1)
    table_a = mask_weight[:_F_SPLIT * VOCAB_PER_FIELD].reshape(-1)
    table_b = mask_weight[_F_SPLIT * VOCAB_PER_FIELD:].reshape(-1)
    mw_a = _sc_gather_a(raw_flat, table_a)
    mw_b = _sc_gather_b(raw_flat, table_b)
    out_t = _tc_mul(xt, mw_a, mw_b)
    return jnp.transpose(out_t, (2, 0, 1))


# speedup vs baseline: 1.1665x; 1.1665x over previous
import functools

import jax
import jax.numpy as jnp
import numpy as np
from jax import lax
from jax.experimental import pallas as pl
from jax.experimental.pallas import tpu as pltpu
from jax.experimental.pallas import tpu_sc as plsc

N_FIELDS = 26
VOCAB_PER_FIELD = 100000
BATCH = 4096
EMBED_DIM = 64
TOTAL_ROWS = N_FIELDS * VOCAB_PER_FIELD
N_IDX = BATCH * N_FIELDS

GAMMA = 2000.0
PRETRAIN_EPOCH = 5
_TEMP = float(GAMMA ** (1.0 / (PRETRAIN_EPOCH - 1)))
_SCALING = float(1.0 + np.exp(-0.5))

_NC = 2
_NS = 16
_NW = _NC * _NS
_LANES = 16

_F_SPLIT = 20


def _make_sc_gather(first_field, n_fields):
    n_idx = n_fields * BATCH
    chunk = n_idx // _NW
    vecs = chunk // _LANES

    def body(raw_hbm, table_hbm, out_hbm, idx_v, rows_v, sem):
        wid = lax.axis_index("s") * _NC + lax.axis_index("c")
        base = wid * chunk
        pltpu.sync_copy(
            raw_hbm.at[pl.ds(first_field * BATCH + base, chunk)], idx_v)

        lane = lax.iota(jnp.int32, _LANES)

        @pl.loop(0, vecs, unroll=8)
        def _(i):
            s = pl.ds(i * _LANES, _LANES)
            pos = base + i * _LANES + lane
            field = lax.shift_right_logical(pos, 12)
            idx_v[s] = idx_v[s] + field * VOCAB_PER_FIELD

        pltpu.async_copy(table_hbm.at[idx_v], rows_v, sem).wait()
        pltpu.sync_copy(rows_v, out_hbm.at[pl.ds(base, chunk)])

    return functools.partial(
        pl.kernel,
        out_type=jax.ShapeDtypeStruct((n_idx,), jnp.float32),
        mesh=plsc.VectorSubcoreMesh(
            core_axis_name="c", subcore_axis_name="s", num_cores=_NC,
            num_subcores=_NS,
        ),
        scratch_types=[
            pltpu.VMEM((chunk,), jnp.int32),
            pltpu.VMEM((chunk,), jnp.float32),
            pltpu.SemaphoreType.DMA,
        ],
    )(body)


_sc_gather_a = _make_sc_gather(0, _F_SPLIT)
_sc_gather_b = _make_sc_gather(_F_SPLIT, N_FIELDS - _F_SPLIT)

_F_BLK = 2
_BLK_SPLIT = _F_SPLIT // _F_BLK


def _tc_mul_body(x_ref, mwa_ref, mwb_ref, o_ref):
    f = pl.program_id(0)
    mw = jnp.where(f < _BLK_SPLIT, mwa_ref[...], mwb_ref[...])
    gate = _SCALING * jax.nn.sigmoid(_TEMP * mw)
    o_ref[...] = x_ref[...] * gate.reshape(_F_BLK, 1, BATCH)


_tc_mul = pl.pallas_call(
    _tc_mul_body,
    grid=(N_FIELDS // _F_BLK,),
    in_specs=[
        pl.BlockSpec((_F_BLK, EMBED_DIM, BATCH), lambda f: (f, 0, 0)),
        pl.BlockSpec((_F_BLK * BATCH,),
                     lambda f: (jnp.minimum(f, _BLK_SPLIT - 1),)),
        pl.BlockSpec((_F_BLK * BATCH,),
                     lambda f: (jnp.maximum(f - _BLK_SPLIT, 0),)),
    ],
    out_specs=pl.BlockSpec((_F_BLK, EMBED_DIM, BATCH), lambda f: (f, 0, 0)),
    out_shape=jax.ShapeDtypeStruct((N_FIELDS, EMBED_DIM, BATCH), jnp.float32),
)


def kernel(x, current_epoch, current_step, raw_data, mask_weight):
    xt = jnp.transpose(x, (1, 2, 0))
    raw_flat = jnp.transpose(raw_data, (1, 0)).astype(jnp.int32).reshape(-1)
    table_a = mask_weight[:_F_SPLIT * VOCAB_PER_FIELD].reshape(-1)
    table_b = mask_weight[_F_SPLIT * VOCAB_PER_FIELD:].reshape(-1)
    mw_a = _sc_gather_a(raw_flat, table_a)
    mw_b = _sc_gather_b(raw_flat, table_b)
    out_t = _tc_mul(xt, mw_a, mw_b)
    return jnp.transpose(out_t, (2, 0, 1))


# speedup vs baseline: 1.9901x; 1.7060x over previous
import functools

import jax
import jax.numpy as jnp
import numpy as np
from jax import lax
from jax.experimental import pallas as pl
from jax.experimental.pallas import tpu as pltpu
from jax.experimental.pallas import tpu_sc as plsc

N_FIELDS = 26
VOCAB_PER_FIELD = 100000
BATCH = 4096
EMBED_DIM = 64
TOTAL_ROWS = N_FIELDS * VOCAB_PER_FIELD
N_IDX = BATCH * N_FIELDS

GAMMA = 2000.0
PRETRAIN_EPOCH = 5
_TEMP = float(GAMMA ** (1.0 / (PRETRAIN_EPOCH - 1)))
_SCALING = float(1.0 + np.exp(-0.5))

_NC = 2
_NS = 16
_NW = _NC * _NS
_LANES = 16

_SLABS = [
    (0, 600000, 0, 6, 0),
    (600000, 600000, 6, 6, 0),
    (1200000, 600000, 12, 6, 0),
    (1800000, 600000, 18, 6, 0),
    (2399360, 200640, 24, 2, 640),
]


def _make_sc_gather(first_field, n_fields, extra):
    n_idx = n_fields * BATCH
    chunk = n_idx // _NW
    vecs = chunk // _LANES

    def body(raw_hbm, table_hbm, out_hbm, idx_v, rows_v, sem):
        wid = lax.axis_index("s") * _NC + lax.axis_index("c")
        base = wid * chunk
        pltpu.sync_copy(
            raw_hbm.at[pl.ds(first_field * BATCH + base, chunk)], idx_v)

        lane = lax.iota(jnp.int32, _LANES)

        @pl.loop(0, vecs, unroll=8)
        def _(i):
            s = pl.ds(i * _LANES, _LANES)
            pos = base + i * _LANES + lane
            field = lax.shift_right_logical(pos, 12)
            idx_v[s] = idx_v[s] + (field * VOCAB_PER_FIELD + extra)

        pltpu.async_copy(table_hbm.at[idx_v], rows_v, sem).wait()
        pltpu.sync_copy(rows_v, out_hbm.at[pl.ds(base, chunk)])

    return functools.partial(
        pl.kernel,
        out_type=jax.ShapeDtypeStruct((n_idx,), jnp.float32),
        mesh=plsc.VectorSubcoreMesh(
            core_axis_name="c", subcore_axis_name="s", num_cores=_NC,
            num_subcores=_NS,
        ),
        scratch_types=[
            pltpu.VMEM((chunk,), jnp.int32),
            pltpu.VMEM((chunk,), jnp.float32),
            pltpu.SemaphoreType.DMA,
        ],
    )(body)


_sc_gathers = [
    _make_sc_gather(ff, nf, extra) for _, _, ff, nf, extra in _SLABS
]

_F_BLK = 2


def _tc_mul_body(x_ref, mw_ref, o_ref):
    gate = _SCALING * jax.nn.sigmoid(_TEMP * mw_ref[...])
    o_ref[...] = x_ref[...] * gate.reshape(_F_BLK, 1, BATCH)


_tc_mul = pl.pallas_call(
    _tc_mul_body,
    grid=(N_FIELDS // _F_BLK,),
    in_specs=[
        pl.BlockSpec((_F_BLK, EMBED_DIM, BATCH), lambda f: (f, 0, 0)),
        pl.BlockSpec((_F_BLK * BATCH,), lambda f: (f,)),
    ],
    out_specs=pl.BlockSpec((_F_BLK, EMBED_DIM, BATCH), lambda f: (f, 0, 0)),
    out_shape=jax.ShapeDtypeStruct((N_FIELDS, EMBED_DIM, BATCH), jnp.float32),
)


def kernel(x, current_epoch, current_step, raw_data, mask_weight):
    xt = jnp.transpose(x, (1, 2, 0))
    raw_flat = jnp.transpose(raw_data, (1, 0)).astype(jnp.int32).reshape(-1)
    gates = []
    for gather, (r0, nrows, _, _, _) in zip(_sc_gathers, _SLABS):
        slab = mask_weight[r0:r0 + nrows].reshape(-1)
        gates.append(gather(raw_flat, slab))
    mw_flat = jnp.concatenate(gates)
    out_t = _tc_mul(xt, mw_flat)
    return jnp.transpose(out_t, (2, 0, 1))


# speedup vs baseline: 2.6215x; 1.3173x over previous
import functools

import jax
import jax.numpy as jnp
import numpy as np
from jax import lax
from jax.experimental import pallas as pl
from jax.experimental.pallas import tpu as pltpu
from jax.experimental.pallas import tpu_sc as plsc

N_FIELDS = 26
VOCAB_PER_FIELD = 100000
BATCH = 4096
EMBED_DIM = 64
TOTAL_ROWS = N_FIELDS * VOCAB_PER_FIELD
N_IDX = BATCH * N_FIELDS

GAMMA = 2000.0
PRETRAIN_EPOCH = 5
_TEMP = float(GAMMA ** (1.0 / (PRETRAIN_EPOCH - 1)))
_SCALING = float(1.0 + np.exp(-0.5))

_NC = 2
_NS = 16
_NW = _NC * _NS
_LANES = 16

_SLABS = [
    (0, 600000, 0, 6, 0),
    (600000, 600000, 6, 6, 0),
    (1200000, 600000, 12, 6, 0),
    (1800000, 600000, 18, 6, 0),
    (2399360, 200640, 24, 2, 640),
]


_VECS = BATCH // _LANES


def _sc_gather_body(raw_hbm, t0, t1, t2, t3, t4, out_hbm, idx_v, rows_v, sem):
    f = lax.axis_index("s") * _NC + lax.axis_index("c")

    @pl.when(f < N_FIELDS)
    def _():
        base = f * BATCH
        pltpu.sync_copy(raw_hbm.at[pl.ds(base, BATCH)], idx_v)

        slab = jnp.minimum(f // 6, 4)
        local = f - slab * 6
        off = jnp.where(f >= 24, local * VOCAB_PER_FIELD + 640,
                        local * VOCAB_PER_FIELD).astype(jnp.int32)

        @pl.loop(0, _VECS, unroll=8)
        def _(i):
            s = pl.ds(i * _LANES, _LANES)
            idx_v[s] = idx_v[s] + off

        for j, tab in enumerate((t0, t1, t2, t3, t4)):
            lo = j * 6

            @pl.when((f >= lo) & (f < min(lo + 6, N_FIELDS)))
            def _(tab=tab):
                pltpu.async_copy(tab.at[idx_v], rows_v, sem).wait()

        pltpu.sync_copy(rows_v, out_hbm.at[pl.ds(base, BATCH)])


_sc_gather = functools.partial(
    pl.kernel,
    out_type=jax.ShapeDtypeStruct((N_IDX,), jnp.float32),
    mesh=plsc.VectorSubcoreMesh(
        core_axis_name="c", subcore_axis_name="s", num_cores=_NC,
        num_subcores=_NS,
    ),
    scratch_types=[
        pltpu.VMEM((BATCH,), jnp.int32),
        pltpu.VMEM((BATCH,), jnp.float32),
        pltpu.SemaphoreType.DMA,
    ],
)(_sc_gather_body)

_F_BLK = 2


def _tc_mul_body(x_ref, mw_ref, o_ref):
    gate = _SCALING * jax.nn.sigmoid(_TEMP * mw_ref[...])
    o_ref[...] = x_ref[...] * gate.reshape(_F_BLK, 1, BATCH)


_tc_mul = pl.pallas_call(
    _tc_mul_body,
    grid=(N_FIELDS // _F_BLK,),
    in_specs=[
        pl.BlockSpec((_F_BLK, EMBED_DIM, BATCH), lambda f: (f, 0, 0)),
        pl.BlockSpec((_F_BLK * BATCH,), lambda f: (f,)),
    ],
    out_specs=pl.BlockSpec((_F_BLK, EMBED_DIM, BATCH), lambda f: (f, 0, 0)),
    out_shape=jax.ShapeDtypeStruct((N_FIELDS, EMBED_DIM, BATCH), jnp.float32),
)


def kernel(x, current_epoch, current_step, raw_data, mask_weight):
    xt = jnp.transpose(x, (1, 2, 0))
    raw_flat = jnp.transpose(raw_data, (1, 0)).astype(jnp.int32).reshape(-1)
    slabs = [mask_weight[r0:r0 + nrows].reshape(-1)
             for r0, nrows, _, _, _ in _SLABS]
    mw_flat = _sc_gather(raw_flat, *slabs)
    out_t = _tc_mul(xt, mw_flat)
    return jnp.transpose(out_t, (2, 0, 1))
